# initial kernel scaffold (unmeasured)
import jax
import jax.numpy as jnp
from jax import lax
from jax.experimental import pallas as pl
from jax.experimental.pallas import tpu as pltpu

N_DEV = 4
FIX_STEPS = 64


def kernel(x, A, B, C):
    Bb, S, D = x.shape
    N = A.shape[-1]

    AT = A.T
    BT = jnp.swapaxes(B, 1, 2)
    CT = jnp.swapaxes(C, 1, 2)

    def body(x_ref, at_ref, bt_ref, ct_ref, out_ref,
             hend_ref, carry_ref, send_sem, recv_sem):
        my = lax.axis_index("i")
        left = (my - 1) % N_DEV
        right = (my + 1) % N_DEV

        barrier = pltpu.get_barrier_semaphore()
        for nbr in (left, right):
            pl.semaphore_signal(
                barrier, inc=1,
                device_id=(nbr,), device_id_type=pl.DeviceIdType.MESH,
            )
        pl.semaphore_wait(barrier, 2)

        dAT = jnp.exp(at_ref[:, :])[None]

        def step(t, h):
            xt = x_ref[:, pl.ds(t, 1), :]
            bt = bt_ref[:, :, pl.ds(t, 1)]
            ct = ct_ref[:, :, pl.ds(t, 1)]
            h = h * dAT + bt * xt
            out_ref[:, pl.ds(t, 1), :] = jnp.sum(h * ct, axis=1, keepdims=True)
            return h

        h0 = jnp.zeros((Bb, N, D), jnp.float32)
        hend_ref[...] = lax.fori_loop(0, S, step, h0)

        rdma = pltpu.make_async_remote_copy(
            src_ref=hend_ref,
            dst_ref=carry_ref,
            send_sem=send_sem,
            recv_sem=recv_sem,
            device_id=(right,),
            device_id_type=pl.DeviceIdType.MESH,
        )
        rdma.start()
        rdma.wait()

        @pl.when(my != 0)
        def _fixup():
            def fstep(t, g):
                g = g * dAT
                ct = ct_ref[:, :, pl.ds(t, 1)]
                dy = jnp.sum(g * ct, axis=1, keepdims=True)
                out_ref[:, pl.ds(t, 1), :] = out_ref[:, pl.ds(t, 1), :] + dy
                return g

            lax.fori_loop(0, FIX_STEPS, fstep, carry_ref[...])

    return pl.pallas_call(
        body,
        out_shape=jax.ShapeDtypeStruct((Bb, S, D), jnp.float32),
        in_specs=[pl.BlockSpec(memory_space=pltpu.VMEM)] * 4,
        out_specs=pl.BlockSpec(memory_space=pltpu.VMEM),
        scratch_shapes=[
            pltpu.VMEM((Bb, N, D), jnp.float32),
            pltpu.VMEM((Bb, N, D), jnp.float32),
            pltpu.SemaphoreType.DMA,
            pltpu.SemaphoreType.DMA,
        ],
        compiler_params=pltpu.CompilerParams(collective_id=0),
    )(x, AT, BT, CT)


# baseline (device time: 41581 ns/iter reference)
import jax
import jax.numpy as jnp
from jax import lax
from jax.experimental import pallas as pl
from jax.experimental.pallas import tpu as pltpu

N_DEV = 4
FIX_STEPS = 64

_OUTER = (((1,), (1,)), ((0,), (0,)))
_INNER = (((2,), (1,)), ((0,), (0,)))


def kernel(x, A, B, C):
    Bb, S, D = x.shape
    N = A.shape[-1]
    AT = A.T

    def body(x_ref, at_ref, b_ref, c_ref, out_ref,
             hend_ref, carry_ref, send_sem, recv_sem):
        my = lax.axis_index("i")
        left = (my - 1) % N_DEV
        right = (my + 1) % N_DEV

        barrier = pltpu.get_barrier_semaphore()
        for nbr in (left, right):
            pl.semaphore_signal(
                barrier, inc=1,
                device_id=(nbr,), device_id_type=pl.DeviceIdType.MESH,
            )
        pl.semaphore_wait(barrier, 2)

        dAT = jnp.exp(at_ref[:, :])[None]

        h = jnp.zeros((Bb, N, D), jnp.float32)
        for t in range(S):
            xt = x_ref[:, t:t + 1, :]
            bt = b_ref[:, t:t + 1, :]
            ct = c_ref[:, t:t + 1, :]
            h = h * dAT + lax.dot_general(
                bt, xt, _OUTER, preferred_element_type=jnp.float32)
            out_ref[:, t:t + 1, :] = lax.dot_general(
                ct, h, _INNER, preferred_element_type=jnp.float32)
        hend_ref[...] = h

        rdma = pltpu.make_async_remote_copy(
            src_ref=hend_ref,
            dst_ref=carry_ref,
            send_sem=send_sem,
            recv_sem=recv_sem,
            device_id=(right,),
            device_id_type=pl.DeviceIdType.MESH,
        )
        rdma.start()
        rdma.wait()

        @pl.when(my != 0)
        def _fixup():
            g = carry_ref[...]
            for t in range(FIX_STEPS):
                g = g * dAT
                ct = c_ref[:, t:t + 1, :]
                dy = lax.dot_general(
                    ct, g, _INNER, preferred_element_type=jnp.float32)
                out_ref[:, t:t + 1, :] = out_ref[:, t:t + 1, :] + dy

    return pl.pallas_call(
        body,
        out_shape=jax.ShapeDtypeStruct((Bb, S, D), jnp.float32),
        in_specs=[pl.BlockSpec(memory_space=pltpu.VMEM)] * 4,
        out_specs=pl.BlockSpec(memory_space=pltpu.VMEM),
        scratch_shapes=[
            pltpu.VMEM((Bb, N, D), jnp.float32),
            pltpu.VMEM((Bb, N, D), jnp.float32),
            pltpu.SemaphoreType.DMA,
            pltpu.SemaphoreType.DMA,
        ],
        compiler_params=pltpu.CompilerParams(collective_id=0),
    )(x, AT, B, C)
